# baseline (speedup 1.0000x reference)
import functools

import jax
import jax.numpy as jnp
from jax import lax
from jax.experimental import pallas as pl
from jax.experimental.pallas import tpu as pltpu
from jax.experimental.pallas import tpu_sc as plsc

_VOCAB = 100000
_EMB = 17
_EPAD = 24
_FIELDS = 26
_LANES = 16


@functools.lru_cache(maxsize=None)
def _make_gather(B):
    NC, NS = 2, 16
    NW = NC * NS
    rows_per_w = B // NW
    R = 64
    N = R * _FIELDS
    NIDX = N // 128
    n_chunks = rows_per_w // R

    mesh = plsc.VectorSubcoreMesh(core_axis_name="c", subcore_axis_name="s")

    @functools.partial(
        pl.kernel,
        mesh=mesh,
        out_type=jax.ShapeDtypeStruct((B * _FIELDS, _EPAD), jnp.float32),
        compiler_params=pltpu.CompilerParams(use_tc_tiling_on_sc=False),
        scratch_types=[
            pltpu.VMEM((N,), jnp.int32),
            pltpu.VMEM((NIDX, 128), jnp.int32),
            pltpu.VMEM((N, _EPAD), jnp.float32),
            pltpu.SemaphoreType.DMA,
        ],
    )
    def k(tables_hbm, xcat_hbm, out_hbm, xcat_v, idx_v, g_v, sem):
        wid = lax.axis_index("s") * NC + lax.axis_index("c")
        lane = lax.iota(jnp.int32, _LANES)

        def chunk_body(c, _):
            cfb = wid * (rows_per_w * _FIELDS) + c * N
            pltpu.sync_copy(xcat_hbm.at[pl.ds(cfb, N)], xcat_v)

            for j in range(NIDX):
                def idx_body(t, _):
                    p = j * 128 + t * _LANES
                    raw = xcat_v[pl.ds(p, _LANES)]
                    f = (p + lane) % _FIELDS
                    val = jnp.clip(raw, 0, _VOCAB - 1) + f * _VOCAB
                    idx_v[j, pl.ds(t * _LANES, _LANES)] = val
                    return 0

                lax.fori_loop(0, 128 // _LANES, idx_body, 0)

            copies = [
                pltpu.async_copy(
                    tables_hbm.at[idx_v.at[j]],
                    g_v.at[pl.ds(j * 128, 128)],
                    sem,
                )
                for j in range(NIDX)
            ]
            for cp in copies:
                cp.wait()

            pltpu.sync_copy(g_v, out_hbm.at[pl.ds(cfb, N)])
            return 0

        lax.fori_loop(0, n_chunks, chunk_body, 0)

    return k


def kernel(x_num, x_cat, tables):
    B = x_cat.shape[0]
    tables_pad = jnp.pad(tables, ((0, 0), (0, 0), (0, _EPAD - _EMB)))
    tables_flat = tables_pad.reshape(_FIELDS * _VOCAB, _EPAD)
    xcat_flat = x_cat.astype(jnp.int32).reshape(B * _FIELDS)
    emb24 = _make_gather(B)(tables_flat, xcat_flat)
    emb = emb24.reshape(B, _FIELDS, _EPAD)[:, :, :_EMB].reshape(B, _FIELDS * _EMB)
    return jnp.concatenate([x_num.astype(jnp.float32), emb], axis=-1)


# speedup vs baseline: 1.4271x; 1.4271x over previous
import functools

import jax
import jax.numpy as jnp
from jax import lax
from jax.experimental import pallas as pl
from jax.experimental.pallas import tpu as pltpu
from jax.experimental.pallas import tpu_sc as plsc

_VOCAB = 100000
_EMB = 17
_FIELDS = 26
_LANES = 16
_PLANE = _FIELDS * _VOCAB


@functools.lru_cache(maxsize=None)
def _make_gather(B):
    NC, NS = 2, 16
    NW = NC * NS
    rows_per_w = B // NW
    R = 64
    N = R * _FIELDS
    NE = N * _EMB
    NT = N // _LANES
    n_chunks = rows_per_w // R
    BF = B * _FIELDS

    mesh = plsc.VectorSubcoreMesh(core_axis_name="c", subcore_axis_name="s")

    @functools.partial(
        pl.kernel,
        mesh=mesh,
        out_type=jax.ShapeDtypeStruct((_EMB * BF,), jnp.float32),
        compiler_params=pltpu.CompilerParams(use_tc_tiling_on_sc=False),
        scratch_types=[
            pltpu.VMEM((N,), jnp.int32),
            pltpu.VMEM((NE,), jnp.int32),
            pltpu.VMEM((NE,), jnp.float32),
            pltpu.SemaphoreType.DMA,
        ],
    )
    def k(t2_hbm, xcat_hbm, out_hbm, xcat_v, idx_v, g_v, sem):
        wid = lax.axis_index("s") * NC + lax.axis_index("c")
        lane = lax.iota(jnp.int32, _LANES)

        def chunk_body(c, _):
            cfb = wid * (rows_per_w * _FIELDS) + c * N
            pltpu.sync_copy(xcat_hbm.at[pl.ds(cfb, N)], xcat_v)

            def idx_body(t, _):
                raw = xcat_v[pl.ds(t * _LANES, _LANES)]
                f = (t * _LANES + lane) % _FIELDS
                tri = jnp.clip(raw, 0, _VOCAB - 1) + f * _VOCAB
                for ee in range(_EMB):
                    idx_v[pl.ds(ee * N + t * _LANES, _LANES)] = tri + ee * _PLANE
                return 0

            lax.fori_loop(0, NT, idx_body, 0)

            pltpu.async_copy(t2_hbm.at[idx_v], g_v, sem).wait()
            for ee in range(_EMB):
                pltpu.sync_copy(
                    g_v.at[pl.ds(ee * N, N)],
                    out_hbm.at[pl.ds(ee * BF + cfb, N)],
                )
            return 0

        lax.fori_loop(0, n_chunks, chunk_body, 0)

    return k


def kernel(x_num, x_cat, tables):
    B = x_cat.shape[0]
    t2 = tables.transpose(2, 0, 1).reshape(_EMB * _PLANE)
    xcat_flat = x_cat.astype(jnp.int32).reshape(B * _FIELDS)
    emb_pm = _make_gather(B)(t2, xcat_flat)
    emb = (
        emb_pm.reshape(_EMB, B, _FIELDS)
        .transpose(1, 2, 0)
        .reshape(B, _FIELDS * _EMB)
    )
    return jnp.concatenate([x_num.astype(jnp.float32), emb], axis=-1)


# speedup vs baseline: 1.4289x; 1.0013x over previous
import functools

import jax
import jax.numpy as jnp
from jax import lax
from jax.experimental import pallas as pl
from jax.experimental.pallas import tpu as pltpu
from jax.experimental.pallas import tpu_sc as plsc

_VOCAB = 100000
_EMB = 17
_FIELDS = 26
_LANES = 16
_PLANE = _FIELDS * _VOCAB


@functools.lru_cache(maxsize=None)
def _make_gather(B):
    NC, NS = 2, 16
    NW = NC * NS
    rows_per_w = B // NW
    R = 64
    N = R * _FIELDS
    NT = N // _LANES
    n_chunks = rows_per_w // R
    BF = B * _FIELDS

    mesh = plsc.VectorSubcoreMesh(core_axis_name="c", subcore_axis_name="s")

    @functools.partial(
        pl.kernel,
        mesh=mesh,
        out_type=jax.ShapeDtypeStruct((_EMB * BF,), jnp.float32),
        compiler_params=pltpu.CompilerParams(use_tc_tiling_on_sc=False),
        scratch_types=[
            pltpu.VMEM((N,), jnp.int32),
            pltpu.VMEM((N,), jnp.int32),
            pltpu.VMEM((_EMB * N,), jnp.float32),
            pltpu.SemaphoreType.DMA,
        ],
    )
    def k(t2_hbm, xcat_hbm, out_hbm, xcat_v, idx_v, g_v, sem):
        wid = lax.axis_index("s") * NC + lax.axis_index("c")
        lane = lax.iota(jnp.int32, _LANES)

        def chunk_body(c, _):
            cfb = wid * (rows_per_w * _FIELDS) + c * N
            pltpu.sync_copy(xcat_hbm.at[pl.ds(cfb, N)], xcat_v)

            def idx_body(t, _):
                raw = xcat_v[pl.ds(t * _LANES, _LANES)]
                f = (t * _LANES + lane) % _FIELDS
                idx_v[pl.ds(t * _LANES, _LANES)] = (
                    jnp.clip(raw, 0, _VOCAB - 1) + f * _VOCAB
                )
                return 0

            lax.fori_loop(0, NT, idx_body, 0)

            copies = [
                pltpu.async_copy(
                    t2_hbm.at[ee].at[idx_v],
                    g_v.at[pl.ds(ee * N, N)],
                    sem,
                )
                for ee in range(_EMB)
            ]
            for cp in copies:
                cp.wait()

            for ee in range(_EMB):
                pltpu.sync_copy(
                    g_v.at[pl.ds(ee * N, N)],
                    out_hbm.at[pl.ds(ee * BF + cfb, N)],
                )
            return 0

        lax.fori_loop(0, n_chunks, chunk_body, 0)

    return k


def kernel(x_num, x_cat, tables):
    B = x_cat.shape[0]
    t2 = tables.transpose(2, 0, 1).reshape(_EMB, _PLANE)
    xcat_flat = x_cat.astype(jnp.int32).reshape(B * _FIELDS)
    emb_pm = _make_gather(B)(t2, xcat_flat)
    emb = (
        emb_pm.reshape(_EMB, B, _FIELDS)
        .transpose(1, 2, 0)
        .reshape(B, _FIELDS * _EMB)
    )
    return jnp.concatenate([x_num.astype(jnp.float32), emb], axis=-1)


# speedup vs baseline: 5.1517x; 3.6053x over previous
import functools

import jax
import jax.numpy as jnp
from jax import lax
from jax.experimental import pallas as pl
from jax.experimental.pallas import tpu as pltpu
from jax.experimental.pallas import tpu_sc as plsc

_VOCAB = 100000
_VPAD = 100096
_VT = _VPAD // 128
_EMB = 17
_FIELDS = 26
_FPAD = 28
_ROWS = _FPAD * _VT
_PSTRIDE = _FPAD * _VPAD
_LANES = 16


def _detile(tables):
    t2b = tables.transpose(2, 0, 1)

    def body(in_ref, out_ref):
        x = in_ref[0]
        xp = jnp.concatenate(
            [x, jnp.zeros((_FIELDS, _VPAD - _VOCAB), jnp.float32)], axis=1
        )
        xp = jnp.concatenate(
            [xp, jnp.zeros((_FPAD - _FIELDS, _VPAD), jnp.float32)], axis=0
        )
        out_ref[...] = xp.reshape(_ROWS, 128)

    out = pl.pallas_call(
        body,
        grid=(_EMB,),
        in_specs=[pl.BlockSpec((1, _FIELDS, _VOCAB), lambda e: (e, 0, 0))],
        out_specs=pl.BlockSpec((_ROWS, 128), lambda e: (e, 0)),
        out_shape=jax.ShapeDtypeStruct((_EMB * _ROWS, 128), jnp.float32),
    )(t2b)
    return out.reshape(_EMB * _ROWS * 128)


@functools.lru_cache(maxsize=None)
def _make_gather(B):
    NC, NS = 2, 16
    NW = NC * NS
    rows_per_w = B // NW
    R = 64
    N = R * _FIELDS
    NT = N // _LANES
    n_chunks = rows_per_w // R
    BF = B * _FIELDS

    mesh = plsc.VectorSubcoreMesh(core_axis_name="c", subcore_axis_name="s")

    @functools.partial(
        pl.kernel,
        mesh=mesh,
        out_type=jax.ShapeDtypeStruct((_EMB * BF,), jnp.float32),
        compiler_params=pltpu.CompilerParams(use_tc_tiling_on_sc=False),
        scratch_types=[
            pltpu.VMEM((N,), jnp.int32),
            pltpu.VMEM((N,), jnp.int32),
            pltpu.VMEM((_EMB * N,), jnp.float32),
            pltpu.SemaphoreType.DMA,
        ],
    )
    def k(t1d_hbm, xcat_hbm, out_hbm, xcat_v, idx_v, g_v, sem):
        wid = lax.axis_index("s") * NC + lax.axis_index("c")
        lane = lax.iota(jnp.int32, _LANES)

        def chunk_body(c, _):
            cfb = wid * (rows_per_w * _FIELDS) + c * N
            pltpu.sync_copy(xcat_hbm.at[pl.ds(cfb, N)], xcat_v)

            def idx_body(t, _):
                raw = xcat_v[pl.ds(t * _LANES, _LANES)]
                f = (t * _LANES + lane) % _FIELDS
                idx_v[pl.ds(t * _LANES, _LANES)] = (
                    jnp.clip(raw, 0, _VOCAB - 1) + f * _VPAD
                )
                return 0

            lax.fori_loop(0, NT, idx_body, 0)

            copies = [
                pltpu.async_copy(
                    t1d_hbm.at[pl.ds(ee * _PSTRIDE, _PSTRIDE)].at[idx_v],
                    g_v.at[pl.ds(ee * N, N)],
                    sem,
                )
                for ee in range(_EMB)
            ]
            for cp in copies:
                cp.wait()

            for ee in range(_EMB):
                pltpu.sync_copy(
                    g_v.at[pl.ds(ee * N, N)],
                    out_hbm.at[pl.ds(ee * BF + cfb, N)],
                )
            return 0

        lax.fori_loop(0, n_chunks, chunk_body, 0)

    return k


def kernel(x_num, x_cat, tables):
    B = x_cat.shape[0]
    t1d = _detile(tables)
    xcat_flat = x_cat.astype(jnp.int32).reshape(B * _FIELDS)
    emb_pm = _make_gather(B)(t1d, xcat_flat)
    emb = (
        emb_pm.reshape(_EMB, B, _FIELDS)
        .transpose(1, 2, 0)
        .reshape(B, _FIELDS * _EMB)
    )
    return jnp.concatenate([x_num.astype(jnp.float32), emb], axis=-1)
